# baseline (speedup 1.0000x reference)
import functools

import jax
import jax.numpy as jnp
from jax.experimental import pallas as pl
from jax.experimental.pallas import tpu as pltpu
from jax.sharding import Mesh, NamedSharding, PartitionSpec as P

BN_EPS = 1e-5
_LANES = 128
_SUB = 8
_LOAD_STREAMS = 8
_STORE_CHUNK = 4
_RAMP = (1, 1, 2, 4, 4)
_STORE_SLOTS = 2


def _pairs(c_in):
    return [(i, j) for i in range(c_in) for j in range(i, c_in)]


def _image_stats(x_img, c_in, pairs):
    r = x_img.shape[1]
    g = r // _SUB

    def fold(a):
        return jnp.sum(a.reshape(g, _SUB, _LANES), axis=0)

    rows = [fold(x_img[i]) for i in range(c_in)]
    rows += [fold(x_img[i] * x_img[j]) for (i, j) in pairs]
    return jnp.stack(rows, axis=0)


def _finalize(stats, w_ref, g_ref, be_ref, c_in, c_out, inv_m, pairs):
    st = jnp.sum(stats, axis=(1, 2))
    s_vec = st[0:c_in]
    w = w_ref[...].astype(jnp.float32)
    mean0 = jnp.sum(w * s_vec[None, :], axis=1) * inv_m
    e2 = jnp.zeros((c_out,), jnp.float32)
    for k, (i, j) in enumerate(pairs):
        coef = 1.0 if i == j else 2.0
        e2 = e2 + (coef * st[c_in + k]) * (w[:, i] * w[:, j])
    var = jnp.maximum(e2 * inv_m - mean0 * mean0, 0.0)
    scale = g_ref[...][:, 0] * jax.lax.rsqrt(var + BN_EPS)
    shift = be_ref[...][:, 0] - mean0 * scale
    wf = w * scale[:, None]
    wf_s = [[wf[c, i] for i in range(c_in)] for c in range(c_out)]
    sh_s = [shift[c] for c in range(c_out)]
    return wf_s, sh_s


def _conv_image(x_img, out_buf, slot, t, c_in, c_out, wf_s, sh_s):
    for c in range(c_out):
        acc = x_img[0] * wf_s[c][0]
        for i in range(1, c_in):
            acc = acc + x_img[i] * wf_s[c][i]
        out_buf[slot, t, c] = jnp.maximum(acc + sh_s[c],
                                          0.0).astype(out_buf.dtype)


def _stats_kernel(c_in, pairs, x_ref, s_ref):
    @pl.when(pl.program_id(0) == 0)
    def _init():
        s_ref[...] = jnp.zeros_like(s_ref)

    s_ref[...] += _image_stats(x_ref[0].astype(jnp.float32), c_in, pairs)


def _apply_kernel(n_loc, c_in, c_out, inv_m, pairs,
                  x_hbm, w_ref, g_ref, be_ref, st_ref, o_hbm,
                  x_vmem, out_buf, ld_sems, st_sems):
    per_l = n_loc // _LOAD_STREAMS if n_loc >= _LOAD_STREAMS else 1
    streams = n_loc // per_l

    ld_copies = []
    for s in range(streams):
        sl = pl.ds(s * per_l, per_l)
        cp = pltpu.make_async_copy(x_hbm.at[sl], x_vmem.at[sl], ld_sems.at[s])
        cp.start()
        ld_copies.append(cp)

    wf_s, sh_s = _finalize(st_ref[...], w_ref, g_ref, be_ref,
                           c_in, c_out, inv_m, pairs)

    for s in range(streams):
        ld_copies[s].wait()
        slot = s % _STORE_SLOTS
        if s >= _STORE_SLOTS:
            buf = out_buf.at[slot, pl.ds(0, per_l)]
            pltpu.make_async_copy(buf, buf, st_sems.at[slot]).wait()
        for t in range(per_l):
            x_img = x_vmem[s * per_l + t].astype(jnp.float32)
            _conv_image(x_img, out_buf, slot, t, c_in, c_out, wf_s, sh_s)
        pltpu.make_async_copy(out_buf.at[slot, pl.ds(0, per_l)],
                              o_hbm.at[pl.ds(s * per_l, per_l)],
                              st_sems.at[slot]).start()

    for di in range(min(_STORE_SLOTS, streams)):
        slot = (streams - 1 - di) % _STORE_SLOTS
        buf = out_buf.at[slot, pl.ds(0, per_l)]
        pltpu.make_async_copy(buf, buf, st_sems.at[slot]).wait()


def _run_shard(x4_loc, w, g_col, be_col, inv_m, n, c_in, c_out, r, pairs):
    n_loc = x4_loc.shape[0]
    k_stats = c_in + len(pairs)
    m_bytes = n_loc * c_in * r * _LANES * 4
    o_bytes = n_loc * c_out * r * _LANES * 4

    partial = pl.pallas_call(
        functools.partial(_stats_kernel, c_in, pairs),
        out_shape=jax.ShapeDtypeStruct((k_stats, _SUB, _LANES), jnp.float32),
        grid=(n_loc,),
        in_specs=[pl.BlockSpec((1, c_in, r, _LANES),
                               lambda j: (j, 0, 0, 0))],
        out_specs=pl.BlockSpec((k_stats, _SUB, _LANES), lambda j: (0, 0, 0)),
        compiler_params=pltpu.CompilerParams(
            dimension_semantics=("arbitrary",),
            vmem_limit_bytes=60000 << 10),
        cost_estimate=pl.CostEstimate(
            flops=3 * n_loc * r * _LANES * c_in * (c_in + 1) // 2,
            transcendentals=0,
            bytes_accessed=m_bytes),
    )(x4_loc)

    stats = jax.lax.psum(partial, "d")

    per_l = n_loc // _LOAD_STREAMS if n_loc >= _LOAD_STREAMS else 1
    streams = n_loc // per_l
    out4_loc = pl.pallas_call(
        functools.partial(_apply_kernel, n_loc, c_in, c_out, inv_m, pairs),
        out_shape=jax.ShapeDtypeStruct((n_loc, c_out, r, _LANES),
                                       x4_loc.dtype),
        grid=(1,),
        in_specs=[
            pl.BlockSpec(memory_space=pl.ANY),
            pl.BlockSpec((c_out, c_in), lambda i: (0, 0)),
            pl.BlockSpec((c_out, 1), lambda i: (0, 0)),
            pl.BlockSpec((c_out, 1), lambda i: (0, 0)),
            pl.BlockSpec((k_stats, _SUB, _LANES), lambda i: (0, 0, 0)),
        ],
        out_specs=pl.BlockSpec(memory_space=pl.ANY),
        scratch_shapes=[
            pltpu.VMEM((n_loc, c_in, r, _LANES), x4_loc.dtype),
            pltpu.VMEM((_STORE_SLOTS, per_l, c_out, r, _LANES),
                       x4_loc.dtype),
            pltpu.SemaphoreType.DMA((streams,)),
            pltpu.SemaphoreType.DMA((_STORE_SLOTS,)),
        ],
        compiler_params=pltpu.CompilerParams(
            dimension_semantics=("arbitrary",),
            vmem_limit_bytes=60000 << 10),
        cost_estimate=pl.CostEstimate(
            flops=2 * n_loc * r * _LANES * c_in * c_out,
            transcendentals=c_out,
            bytes_accessed=m_bytes + o_bytes),
    )(x4_loc, w, g_col, be_col, stats)
    return out4_loc


def _fused_kernel(n, c_in, c_out, inv_m, pairs,
                  x_hbm, w_ref, g_ref, be_ref, o_hbm,
                  x_vmem, out_buf, ld_sems, st_sems):
    per_l = n // _LOAD_STREAMS

    ld_copies = []
    for s in range(_LOAD_STREAMS):
        sl = pl.ds(s * per_l, per_l)
        cp = pltpu.make_async_copy(x_hbm.at[sl], x_vmem.at[sl], ld_sems.at[s])
        cp.start()
        ld_copies.append(cp)

    k_stats = c_in + len(pairs)
    stats = jnp.zeros((k_stats, _SUB, _LANES), jnp.float32)
    for s in range(_LOAD_STREAMS):
        ld_copies[s].wait()

        def sbody(i, acc, base=s * per_l):
            x_img = x_vmem[base + i].astype(jnp.float32)
            return acc + _image_stats(x_img, c_in, pairs)

        stats = jax.lax.fori_loop(0, per_l, sbody, stats)

    wf_s, sh_s = _finalize(stats, w_ref, g_ref, be_ref,
                           c_in, c_out, inv_m, pairs)

    def start_store(slot, start, size):
        pltpu.make_async_copy(out_buf.at[slot, pl.ds(0, size)],
                              o_hbm.at[pl.ds(start, size)],
                              st_sems.at[slot]).start()

    def wait_store(slot, size):
        buf = out_buf.at[slot, pl.ds(0, size)]
        pltpu.make_async_copy(buf, buf, st_sems.at[slot]).wait()

    base = 0
    for ci, size in enumerate(_RAMP):
        slot = ci % _STORE_SLOTS
        if ci >= _STORE_SLOTS:
            wait_store(slot, _RAMP[ci - _STORE_SLOTS])
        for t in range(size):
            x_img = x_vmem[base + t].astype(jnp.float32)
            _conv_image(x_img, out_buf, slot, t, c_in, c_out, wf_s, sh_s)
        start_store(slot, base, size)
        base += size
    ramp_imgs = base

    n_steady = (n - ramp_imgs) // _STORE_CHUNK
    slot0 = len(_RAMP) % _STORE_SLOTS
    last_sizes = [_RAMP[len(_RAMP) - _STORE_SLOTS + k] for k
                  in range(_STORE_SLOTS)]

    def abody(gi, _):
        slot = jax.lax.rem(gi + slot0, _STORE_SLOTS)
        wait_store(slot, _STORE_CHUNK)
        jst = ramp_imgs + gi * _STORE_CHUNK
        for t in range(_STORE_CHUNK):
            x_img = x_vmem[jst + t].astype(jnp.float32)
            _conv_image(x_img, out_buf, slot, t, c_in, c_out, wf_s, sh_s)
        pltpu.make_async_copy(out_buf.at[slot, pl.ds(0, _STORE_CHUNK)],
                              o_hbm.at[pl.ds(jst, _STORE_CHUNK)],
                              st_sems.at[slot]).start()
        return _

    peel = min(_STORE_SLOTS, n_steady)
    for gi in range(peel):
        slot = (gi + slot0) % _STORE_SLOTS
        wait_store(slot, last_sizes[gi])
        jst = ramp_imgs + gi * _STORE_CHUNK
        for t in range(_STORE_CHUNK):
            x_img = x_vmem[jst + t].astype(jnp.float32)
            _conv_image(x_img, out_buf, slot, t, c_in, c_out, wf_s, sh_s)
        start_store(slot, jst, _STORE_CHUNK)
    if n_steady > peel:
        jax.lax.fori_loop(peel, n_steady, abody, 0)

    for di in range(_STORE_SLOTS):
        slot = (n_steady + di + slot0) % _STORE_SLOTS
        wait_store(slot, _STORE_CHUNK if n_steady >= _STORE_SLOTS
                   else last_sizes[min(di + n_steady, len(last_sizes) - 1)])


def _kernel_single(x4, w, g_col, be_col, n, c_in, c_out, r, m, pairs):
    k_stats = c_in + len(pairs)
    x_bytes = n * c_in * r * _LANES * 4
    out_bytes = n * c_out * r * _LANES * 4
    return pl.pallas_call(
        functools.partial(_fused_kernel, n, c_in, c_out, 1.0 / m, pairs),
        out_shape=jax.ShapeDtypeStruct((n, c_out, r, _LANES), x4.dtype),
        grid=(1,),
        in_specs=[
            pl.BlockSpec(memory_space=pl.ANY),
            pl.BlockSpec((c_out, c_in), lambda i: (0, 0)),
            pl.BlockSpec((c_out, 1), lambda i: (0, 0)),
            pl.BlockSpec((c_out, 1), lambda i: (0, 0)),
        ],
        out_specs=pl.BlockSpec(memory_space=pl.ANY),
        scratch_shapes=[
            pltpu.VMEM((n, c_in, r, _LANES), x4.dtype),
            pltpu.VMEM((_STORE_SLOTS, _STORE_CHUNK, c_out, r, _LANES),
                       x4.dtype),
            pltpu.SemaphoreType.DMA((_LOAD_STREAMS,)),
            pltpu.SemaphoreType.DMA((_STORE_SLOTS,)),
        ],
        compiler_params=pltpu.CompilerParams(
            dimension_semantics=("arbitrary",),
            vmem_limit_bytes=60000 << 10),
        cost_estimate=pl.CostEstimate(
            flops=3 * m * c_in * (c_in + 1) // 2 + 2 * m * c_in * c_out
            + 2 * m * c_out,
            transcendentals=c_out,
            bytes_accessed=x_bytes + out_bytes),
    )(x4, w, g_col, be_col)


def kernel(x, weight, bias, gamma, beta):
    n, c_in, h, w_sp = x.shape
    c_out = weight.shape[0]
    hw = h * w_sp
    m = n * hw
    pairs = _pairs(c_in)

    assert hw % (_SUB * _LANES) == 0
    r = hw // _LANES
    x4 = x.reshape(n, c_in, r, _LANES)
    w_f32 = weight.astype(jnp.float32)
    g_col = gamma.astype(jnp.float32).reshape(c_out, 1)
    be_col = beta.astype(jnp.float32).reshape(c_out, 1)

    devs = jax.devices()[:2]
    use_shard = (len(devs) == 2 and n % 2 == 0
                 and (n // 2) % _LOAD_STREAMS == 0)

    if use_shard:
        mesh = Mesh(devs, ("d",))
        shd = NamedSharding(mesh, P("d"))
        rep = NamedSharding(mesh, P())
        x4_s = jax.device_put(x4, shd)
        w_s = jax.device_put(w_f32, rep)
        g_s = jax.device_put(g_col, rep)
        be_s = jax.device_put(be_col, rep)
        body = functools.partial(_run_shard, inv_m=1.0 / m, n=n, c_in=c_in,
                                 c_out=c_out, r=r, pairs=pairs)
        out4 = jax.shard_map(body, mesh=mesh,
                             in_specs=(P("d"), P(), P(), P()),
                             out_specs=P("d"),
                             check_vma=False)(x4_s, w_s, g_s, be_s)
    else:
        assert n % _LOAD_STREAMS == 0
        assert n >= sum(_RAMP) + _STORE_SLOTS * _STORE_CHUNK
        assert (n - sum(_RAMP)) % _STORE_CHUNK == 0
        out4 = _kernel_single(x4, w_f32, g_col, be_col,
                              n, c_in, c_out, r, m, pairs)

    return out4.reshape(n, c_out, h, w_sp)


# speedup vs baseline: 3.6083x; 3.6083x over previous
import functools

import jax
import jax.numpy as jnp
from jax.experimental import pallas as pl
from jax.experimental.pallas import tpu as pltpu

BN_EPS = 1e-5
_LANES = 128
_SUB = 8
_LOAD_STREAMS = 16
_STORE_CHUNK = 4
_RAMP = (1, 1, 2, 4, 4)
_STORE_SLOTS = 2


def _pairs(c_in):
    return [(i, j) for i in range(c_in) for j in range(i, c_in)]


def _image_stats(x_img, c_in, pairs):
    r = x_img.shape[1]
    g = r // _SUB

    def fold(a):
        return jnp.sum(a.reshape(g, _SUB, _LANES), axis=0)

    rows = [fold(x_img[i]) for i in range(c_in)]
    rows += [fold(x_img[i] * x_img[j]) for (i, j) in pairs]
    return jnp.stack(rows, axis=0)


def _fused_kernel(n, c_in, c_out, inv_m, pairs,
                  x_hbm, w_ref, g_ref, be_ref, o_hbm,
                  x_vmem, out_buf, ld_sems, st_sems):
    k_stats = c_in + len(pairs)
    per_l = n // _LOAD_STREAMS

    ld_copies = []
    for s in range(_LOAD_STREAMS):
        sl = pl.ds(s * per_l, per_l)
        cp = pltpu.make_async_copy(x_hbm.at[sl], x_vmem.at[sl], ld_sems.at[s])
        cp.start()
        ld_copies.append(cp)

    stats = jnp.zeros((k_stats, _SUB, _LANES), jnp.float32)
    for s in range(_LOAD_STREAMS):
        ld_copies[s].wait()

        def sbody(i, acc, base=s * per_l):
            x_img = x_vmem[base + i].astype(jnp.float32)
            return acc + _image_stats(x_img, c_in, pairs)

        stats = jax.lax.fori_loop(0, per_l, sbody, stats)

    st = jnp.sum(stats, axis=(1, 2))
    s_vec = st[0:c_in]
    w = w_ref[...].astype(jnp.float32)
    mean0 = jnp.sum(w * s_vec[None, :], axis=1) * inv_m
    e2 = jnp.zeros((c_out,), jnp.float32)
    for k, (i, j) in enumerate(pairs):
        coef = 1.0 if i == j else 2.0
        e2 = e2 + (coef * st[c_in + k]) * (w[:, i] * w[:, j])
    var = jnp.maximum(e2 * inv_m - mean0 * mean0, 0.0)
    scale = g_ref[...][:, 0] * jax.lax.rsqrt(var + BN_EPS)
    shift = be_ref[...][:, 0] - mean0 * scale
    wf = w * scale[:, None]
    wf_s = [[wf[c, i] for i in range(c_in)] for c in range(c_out)]
    sh_s = [shift[c] for c in range(c_out)]

    def conv_image(j, slot, t):
        x_img = x_vmem[j].astype(jnp.float32)
        for c in range(c_out):
            acc = x_img[0] * wf_s[c][0]
            for i in range(1, c_in):
                acc = acc + x_img[i] * wf_s[c][i]
            out_buf[slot, t, c] = jnp.maximum(acc + sh_s[c],
                                              0.0).astype(out_buf.dtype)

    def start_store(slot, start, size):
        pltpu.make_async_copy(out_buf.at[slot, pl.ds(0, size)],
                              o_hbm.at[pl.ds(start, size)],
                              st_sems.at[slot]).start()

    def wait_store(slot, size):
        buf = out_buf.at[slot, pl.ds(0, size)]
        pltpu.make_async_copy(buf, buf, st_sems.at[slot]).wait()

    base = 0
    for ci, size in enumerate(_RAMP):
        slot = ci % _STORE_SLOTS
        if ci >= _STORE_SLOTS:
            wait_store(slot, _RAMP[ci - _STORE_SLOTS])
        for t in range(size):
            conv_image(base + t, slot, t)
        start_store(slot, base, size)
        base += size
    ramp_imgs = base

    n_steady = (n - ramp_imgs) // _STORE_CHUNK
    slot0 = len(_RAMP) % _STORE_SLOTS
    last_sizes = [_RAMP[len(_RAMP) - _STORE_SLOTS + k] for k
                  in range(_STORE_SLOTS)]

    def abody(gi, _):
        slot = jax.lax.rem(gi + slot0, _STORE_SLOTS)
        wait_store(slot, _STORE_CHUNK)
        jst = ramp_imgs + gi * _STORE_CHUNK
        for t in range(_STORE_CHUNK):
            conv_image(jst + t, slot, t)
        pltpu.make_async_copy(out_buf.at[slot, pl.ds(0, _STORE_CHUNK)],
                              o_hbm.at[pl.ds(jst, _STORE_CHUNK)],
                              st_sems.at[slot]).start()
        return _

    peel = min(_STORE_SLOTS, n_steady)
    for gi in range(peel):
        slot = (gi + slot0) % _STORE_SLOTS
        wait_store(slot, last_sizes[gi])
        jst = ramp_imgs + gi * _STORE_CHUNK
        for t in range(_STORE_CHUNK):
            conv_image(jst + t, slot, t)
        start_store(slot, jst, _STORE_CHUNK)
    if n_steady > peel:
        jax.lax.fori_loop(peel, n_steady, abody, 0)

    for di in range(_STORE_SLOTS):
        slot = (n_steady + di + slot0) % _STORE_SLOTS
        if n_steady >= _STORE_SLOTS:
            wait_store(slot, _STORE_CHUNK)
        else:
            wait_store(slot, _STORE_CHUNK if di + n_steady >= len(last_sizes)
                       else last_sizes[di + n_steady])


def kernel(x, weight, bias, gamma, beta):
    n, c_in, h, w_sp = x.shape
    c_out = weight.shape[0]
    hw = h * w_sp
    m = n * hw
    pairs = _pairs(c_in)
    k_stats = c_in + len(pairs)

    assert hw % (_SUB * _LANES) == 0
    assert n % _LOAD_STREAMS == 0
    assert n >= sum(_RAMP) + _STORE_SLOTS * _STORE_CHUNK
    assert (n - sum(_RAMP)) % _STORE_CHUNK == 0
    r = hw // _LANES
    x4 = x.reshape(n, c_in, r, _LANES)

    x_bytes = n * c_in * hw * 4
    out_bytes = n * c_out * hw * 4

    g_col = gamma.astype(jnp.float32).reshape(c_out, 1)
    be_col = beta.astype(jnp.float32).reshape(c_out, 1)

    out4 = pl.pallas_call(
        functools.partial(_fused_kernel, n, c_in, c_out, 1.0 / m, pairs),
        out_shape=jax.ShapeDtypeStruct((n, c_out, r, _LANES), x.dtype),
        grid=(1,),
        in_specs=[
            pl.BlockSpec(memory_space=pl.ANY),
            pl.BlockSpec((c_out, c_in), lambda i: (0, 0)),
            pl.BlockSpec((c_out, 1), lambda i: (0, 0)),
            pl.BlockSpec((c_out, 1), lambda i: (0, 0)),
        ],
        out_specs=pl.BlockSpec(memory_space=pl.ANY),
        scratch_shapes=[
            pltpu.VMEM((n, c_in, r, _LANES), x.dtype),
            pltpu.VMEM((_STORE_SLOTS, _STORE_CHUNK, c_out, r, _LANES),
                       x.dtype),
            pltpu.SemaphoreType.DMA((_LOAD_STREAMS,)),
            pltpu.SemaphoreType.DMA((_STORE_SLOTS,)),
        ],
        compiler_params=pltpu.CompilerParams(
            dimension_semantics=("arbitrary",),
            vmem_limit_bytes=60000 << 10),
        cost_estimate=pl.CostEstimate(
            flops=3 * m * c_in * (c_in + 1) // 2 + 2 * m * c_in * c_out
            + 2 * m * c_out,
            transcendentals=c_out,
            bytes_accessed=x_bytes + out_bytes),
    )(x4, weight.astype(jnp.float32), g_col, be_col)

    return out4.reshape(n, c_out, h, w_sp)


# speedup vs baseline: 3.6499x; 1.0115x over previous
import functools

import jax
import jax.numpy as jnp
from jax.experimental import pallas as pl
from jax.experimental.pallas import tpu as pltpu

BN_EPS = 1e-5
_LANES = 128
_SUB = 8
_LOAD_STREAMS = 8
_STORE_CHUNK = 2
_RAMP = (1, 1, 2, 2)
_STORE_SLOTS = 4


def _pairs(c_in):
    return [(i, j) for i in range(c_in) for j in range(i, c_in)]


def _image_stats(x_img, c_in, pairs):
    r = x_img.shape[1]
    g = r // _SUB

    def fold(a):
        return jnp.sum(a.reshape(g, _SUB, _LANES), axis=0)

    rows = [fold(x_img[i]) for i in range(c_in)]
    rows += [fold(x_img[i] * x_img[j]) for (i, j) in pairs]
    return jnp.stack(rows, axis=0)


def _fused_kernel(n, c_in, c_out, inv_m, pairs,
                  x_hbm, w_ref, g_ref, be_ref, o_hbm,
                  x_vmem, out_buf, ld_sems, st_sems):
    k_stats = c_in + len(pairs)
    per_l = n // _LOAD_STREAMS

    ld_copies = []
    for s in range(_LOAD_STREAMS):
        sl = pl.ds(s * per_l, per_l)
        cp = pltpu.make_async_copy(x_hbm.at[sl], x_vmem.at[sl], ld_sems.at[s])
        cp.start()
        ld_copies.append(cp)

    stats = jnp.zeros((k_stats, _SUB, _LANES), jnp.float32)
    for s in range(_LOAD_STREAMS):
        ld_copies[s].wait()

        def sbody(i, acc, base=s * per_l):
            x_img = x_vmem[base + i].astype(jnp.float32)
            return acc + _image_stats(x_img, c_in, pairs)

        stats = jax.lax.fori_loop(0, per_l, sbody, stats)

    st = jnp.sum(stats, axis=(1, 2))
    s_vec = st[0:c_in]
    w = w_ref[...].astype(jnp.float32)
    mean0 = jnp.sum(w * s_vec[None, :], axis=1) * inv_m
    e2 = jnp.zeros((c_out,), jnp.float32)
    for k, (i, j) in enumerate(pairs):
        coef = 1.0 if i == j else 2.0
        e2 = e2 + (coef * st[c_in + k]) * (w[:, i] * w[:, j])
    var = jnp.maximum(e2 * inv_m - mean0 * mean0, 0.0)
    scale = g_ref[...][:, 0] * jax.lax.rsqrt(var + BN_EPS)
    shift = be_ref[...][:, 0] - mean0 * scale
    wf = w * scale[:, None]
    wf_s = [[wf[c, i] for i in range(c_in)] for c in range(c_out)]
    sh_s = [shift[c] for c in range(c_out)]

    def conv_image(j, slot, t):
        x_img = x_vmem[j].astype(jnp.float32)
        for c in range(c_out):
            acc = x_img[0] * wf_s[c][0]
            for i in range(1, c_in):
                acc = acc + x_img[i] * wf_s[c][i]
            out_buf[slot, t, c] = jnp.maximum(acc + sh_s[c],
                                              0.0).astype(out_buf.dtype)

    def start_store(slot, start, size):
        pltpu.make_async_copy(out_buf.at[slot, pl.ds(0, size)],
                              o_hbm.at[pl.ds(start, size)],
                              st_sems.at[slot]).start()

    def wait_store(slot, size):
        buf = out_buf.at[slot, pl.ds(0, size)]
        pltpu.make_async_copy(buf, buf, st_sems.at[slot]).wait()

    base = 0
    for ci, size in enumerate(_RAMP):
        slot = ci % _STORE_SLOTS
        if ci >= _STORE_SLOTS:
            wait_store(slot, _RAMP[ci - _STORE_SLOTS])
        for t in range(size):
            conv_image(base + t, slot, t)
        start_store(slot, base, size)
        base += size
    ramp_imgs = base

    n_steady = (n - ramp_imgs) // _STORE_CHUNK
    slot0 = len(_RAMP) % _STORE_SLOTS
    last_sizes = [_RAMP[len(_RAMP) - _STORE_SLOTS + k] for k
                  in range(_STORE_SLOTS)]

    def abody(gi, _):
        slot = jax.lax.rem(gi + slot0, _STORE_SLOTS)
        wait_store(slot, _STORE_CHUNK)
        jst = ramp_imgs + gi * _STORE_CHUNK
        for t in range(_STORE_CHUNK):
            conv_image(jst + t, slot, t)
        pltpu.make_async_copy(out_buf.at[slot, pl.ds(0, _STORE_CHUNK)],
                              o_hbm.at[pl.ds(jst, _STORE_CHUNK)],
                              st_sems.at[slot]).start()
        return _

    peel = min(_STORE_SLOTS, n_steady)
    for gi in range(peel):
        slot = (gi + slot0) % _STORE_SLOTS
        wait_store(slot, last_sizes[gi])
        jst = ramp_imgs + gi * _STORE_CHUNK
        for t in range(_STORE_CHUNK):
            conv_image(jst + t, slot, t)
        start_store(slot, jst, _STORE_CHUNK)
    if n_steady > peel:
        jax.lax.fori_loop(peel, n_steady, abody, 0)

    for di in range(_STORE_SLOTS):
        slot = (n_steady + di + slot0) % _STORE_SLOTS
        if n_steady >= _STORE_SLOTS:
            wait_store(slot, _STORE_CHUNK)
        else:
            wait_store(slot, _STORE_CHUNK if di + n_steady >= len(last_sizes)
                       else last_sizes[di + n_steady])


def kernel(x, weight, bias, gamma, beta):
    n, c_in, h, w_sp = x.shape
    c_out = weight.shape[0]
    hw = h * w_sp
    m = n * hw
    pairs = _pairs(c_in)
    k_stats = c_in + len(pairs)

    assert hw % (_SUB * _LANES) == 0
    assert n % _LOAD_STREAMS == 0
    assert n >= sum(_RAMP) + _STORE_SLOTS * _STORE_CHUNK
    assert (n - sum(_RAMP)) % _STORE_CHUNK == 0
    r = hw // _LANES
    x4 = x.reshape(n, c_in, r, _LANES)

    x_bytes = n * c_in * hw * 4
    out_bytes = n * c_out * hw * 4

    g_col = gamma.astype(jnp.float32).reshape(c_out, 1)
    be_col = beta.astype(jnp.float32).reshape(c_out, 1)

    out4 = pl.pallas_call(
        functools.partial(_fused_kernel, n, c_in, c_out, 1.0 / m, pairs),
        out_shape=jax.ShapeDtypeStruct((n, c_out, r, _LANES), x.dtype),
        grid=(1,),
        in_specs=[
            pl.BlockSpec(memory_space=pl.ANY),
            pl.BlockSpec((c_out, c_in), lambda i: (0, 0)),
            pl.BlockSpec((c_out, 1), lambda i: (0, 0)),
            pl.BlockSpec((c_out, 1), lambda i: (0, 0)),
        ],
        out_specs=pl.BlockSpec(memory_space=pl.ANY),
        scratch_shapes=[
            pltpu.VMEM((n, c_in, r, _LANES), x.dtype),
            pltpu.VMEM((_STORE_SLOTS, _STORE_CHUNK, c_out, r, _LANES),
                       x.dtype),
            pltpu.SemaphoreType.DMA((_LOAD_STREAMS,)),
            pltpu.SemaphoreType.DMA((_STORE_SLOTS,)),
        ],
        compiler_params=pltpu.CompilerParams(
            dimension_semantics=("arbitrary",),
            vmem_limit_bytes=60000 << 10),
        cost_estimate=pl.CostEstimate(
            flops=3 * m * c_in * (c_in + 1) // 2 + 2 * m * c_in * c_out
            + 2 * m * c_out,
            transcendentals=c_out,
            bytes_accessed=x_bytes + out_bytes),
    )(x4, weight.astype(jnp.float32), g_col, be_col)

    return out4.reshape(n, c_out, h, w_sp)
